# baseline (speedup 1.0000x reference)
import functools

import jax
import jax.numpy as jnp
from jax import lax
from jax.experimental import pallas as pl
from jax.experimental.pallas import tpu as pltpu
from jax.experimental.pallas import tpu_sc as plsc

B, S, MAXC = 1024, 200, 16
DW, DC, DOUT = 128, 64, 128
N = B * S
NC, NS = 2, 16
NW = NC * NS
TPW = N // NW
C = 128
NCHUNK = TPW // C
VEC = 16
NGRP = DOUT // 32

_PERM = tuple(
    32 * (p // 32) + (16 if p % 2 else 0) + (p % 32) // 2 for p in range(DOUT)
)


def _proj_f32_body(t_ref, w_ref, b_ref, out_ref):
    out_ref[...] = (
        jnp.dot(t_ref[...], w_ref[...], preferred_element_type=jnp.float32)
        + b_ref[...]
    )


def _proj_bf16_body(t_ref, w_ref, b_ref, out_ref):
    acc = (
        jnp.dot(t_ref[...], w_ref[...], preferred_element_type=jnp.float32)
        + b_ref[...]
    )
    out_ref[...] = acc.astype(jnp.bfloat16)


def _project(table, w, b2d, bm, body, out_cols, out_dtype):
    m, k = table.shape
    return pl.pallas_call(
        body,
        grid=(m // bm,),
        in_specs=[
            pl.BlockSpec((bm, k), lambda i: (i, 0)),
            pl.BlockSpec((k, DOUT), lambda i: (0, 0)),
            pl.BlockSpec((1, DOUT), lambda i: (0, 0)),
        ],
        out_specs=pl.BlockSpec((bm, out_cols), lambda i: (i, 0)),
        out_shape=jax.ShapeDtypeStruct((m, out_cols), out_dtype),
    )(table, w, b2d)


def _chunk_compute(pc_v, cid_v, len_v, rows_v, orow_v, offs):
    def grp_body(tg, carry2):
        t0 = tg * VEC
        lens = len_v[pl.ds(t0, VEC)]
        for k in range(VEC):
            t = t0 + k
            nchars = lens[k]
            cvec = cid_v[t, :]
            accs = []
            for c in range(NGRP):
                accs.append(rows_v[t, pl.ds(32 * c, VEC)])
                accs.append(rows_v[t, pl.ds(32 * c + VEC, VEC)])
            accs = tuple(accs)

            def char_body(m, a):
                j0 = 2 * m
                rv0 = cvec.at[jnp.full((VEC,), 0, jnp.int32) + j0].get(
                    mode="promise_in_bounds")
                rv1 = cvec.at[jnp.full((VEC,), 1, jnp.int32) + j0].get(
                    mode="promise_in_bounds")
                out = []
                for c in range(NGRP):
                    gi0 = plsc.load_gather(pc_v, [rv0, offs[c]])
                    gi1 = plsc.load_gather(pc_v, [rv1, offs[c]])
                    da0, db0 = plsc.unpack(
                        plsc.bitcast(gi0, jnp.bfloat16),
                        format=plsc.PackFormat.INTERLEAVED)
                    da1, db1 = plsc.unpack(
                        plsc.bitcast(gi1, jnp.bfloat16),
                        format=plsc.PackFormat.INTERLEAVED)
                    out.append(a[2 * c] + da0 + da1)
                    out.append(a[2 * c + 1] + db0 + db1)
                return tuple(out)

            accs = lax.fori_loop(0, (nchars + 1) // 2, char_body, accs)
            for c in range(NGRP):
                orow_v[t, pl.ds(32 * c, VEC)] = accs[2 * c]
                orow_v[t, pl.ds(32 * c + VEC, VEC)] = accs[2 * c + 1]
        return carry2

    lax.fori_loop(0, C // VEC, grp_body, 0)


@functools.partial(
    pl.kernel,
    out_type=jax.ShapeDtypeStruct((N, DOUT), jnp.float32),
    mesh=plsc.VectorSubcoreMesh(core_axis_name="c", subcore_axis_name="s"),
    scratch_types=[
        pltpu.VMEM((257, DOUT // 2), jnp.int32),
        pltpu.VMEM((C,), jnp.int32),
        pltpu.VMEM((C,), jnp.int32),
        pltpu.VMEM((C, MAXC), jnp.int32),
        pltpu.VMEM((C,), jnp.int32),
        pltpu.VMEM((C, DOUT), jnp.float32),
        pltpu.VMEM((C, DOUT), jnp.float32),
        pltpu.VMEM((C, DOUT), jnp.float32),
        pltpu.VMEM((C, DOUT), jnp.float32),
        pltpu.SemaphoreType.DMA,
        pltpu.SemaphoreType.DMA,
        pltpu.SemaphoreType.DMA,
        pltpu.SemaphoreType.DMA,
    ],
    compiler_params=pltpu.CompilerParams(needs_layout_passes=False),
)
def _sc_embed(pw_hbm, pc_hbm, widx_hbm, cid_hbm, len_hbm, out_hbm,
              pc_v, idx0, idx1, cid_v, len_v,
              rows0, rows1, orow0, orow1, sem0, sem1, semw0, semw1):
    wid = lax.axis_index("s") * NC + lax.axis_index("c")
    base0 = wid * TPW
    pltpu.sync_copy(pc_hbm, pc_v)
    lane = lax.iota(jnp.int32, VEC)
    offs = [lane + VEC * c for c in range(NGRP)]

    def pair_body(i, carry):
        ba = base0 + 2 * i * C
        bb = ba + C
        pltpu.sync_copy(widx_hbm.at[pl.ds(ba, C)], idx0)
        h0 = pltpu.async_copy(pw_hbm.at[idx0], rows0, sem0)
        pltpu.sync_copy(widx_hbm.at[pl.ds(bb, C)], idx1)
        h1 = pltpu.async_copy(pw_hbm.at[idx1], rows1, sem1)
        pltpu.sync_copy(cid_hbm.at[pl.ds(ba, C)], cid_v)
        pltpu.sync_copy(len_hbm.at[pl.ds(ba, C)], len_v)
        h0.wait()
        _chunk_compute(pc_v, cid_v, len_v, rows0, orow0, offs)
        w0 = pltpu.async_copy(orow0, out_hbm.at[pl.ds(ba, C)], semw0)
        pltpu.sync_copy(cid_hbm.at[pl.ds(bb, C)], cid_v)
        pltpu.sync_copy(len_hbm.at[pl.ds(bb, C)], len_v)
        h1.wait()
        _chunk_compute(pc_v, cid_v, len_v, rows1, orow1, offs)
        w1 = pltpu.async_copy(orow1, out_hbm.at[pl.ds(bb, C)], semw1)
        w0.wait()
        w1.wait()
        return carry

    lax.fori_loop(0, NCHUNK // 2, pair_body, 0)


def kernel(word_inputs, char_ids, char_lengths, word_table, char_table, W, b):
    perm = jnp.array(_PERM, dtype=jnp.int32)
    wp = W[:, perm]
    pw = _project(word_table, W[:DW], b.reshape(1, DOUT), 1000,
                  _proj_f32_body, DOUT, jnp.float32)
    pc_bf = _project(char_table, wp[DW:], jnp.zeros((1, DOUT), jnp.float32),
                     256, _proj_bf16_body, DOUT, jnp.bfloat16)
    pc_i32 = lax.bitcast_convert_type(
        pc_bf.reshape(-1, DOUT // 2, 2), jnp.int32)
    pc_i32 = jnp.pad(pc_i32, ((0, 1), (0, 0)))
    widx = word_inputs.reshape(N).astype(jnp.int32)
    cid = char_ids.reshape(N, MAXC).astype(jnp.int32)
    clen = char_lengths.reshape(N).astype(jnp.int32)
    cid = jnp.where(jnp.arange(MAXC, dtype=jnp.int32)[None, :]
                    < clen[:, None], cid, 256)
    out = _sc_embed(pw, pc_i32, widx, cid, clen)
    return out.reshape(B, S, DOUT)


# speedup vs baseline: 2.1961x; 2.1961x over previous
import functools

import jax
import jax.numpy as jnp
from jax import lax
from jax.experimental import pallas as pl
from jax.experimental.pallas import tpu as pltpu
from jax.experimental.pallas import tpu_sc as plsc

B, S, MAXC = 1024, 200, 16
DW, DC, DOUT = 128, 64, 128
N = B * S
NC, NS = 2, 16
NW = NC * NS
TPW = N // NW
C = 128
NCHUNK = TPW // C
VEC = 16
NGRP = DOUT // 32

_PERM = tuple(
    32 * (p // 32) + (16 if p % 2 else 0) + (p % 32) // 2 for p in range(DOUT)
)


def _proj_f32_body(t_ref, w_ref, b_ref, out_ref):
    out_ref[...] = (
        jnp.dot(t_ref[...], w_ref[...], preferred_element_type=jnp.float32)
        + b_ref[...]
    )


def _proj_bf16_body(t_ref, w_ref, b_ref, out_ref):
    acc = (
        jnp.dot(t_ref[...], w_ref[...], preferred_element_type=jnp.float32)
        + b_ref[...]
    )
    out_ref[...] = acc.astype(jnp.bfloat16)


def _project(table, w, b2d, bm, body, out_cols, out_dtype):
    m, k = table.shape
    return pl.pallas_call(
        body,
        grid=(m // bm,),
        in_specs=[
            pl.BlockSpec((bm, k), lambda i: (i, 0)),
            pl.BlockSpec((k, DOUT), lambda i: (0, 0)),
            pl.BlockSpec((1, DOUT), lambda i: (0, 0)),
        ],
        out_specs=pl.BlockSpec((bm, out_cols), lambda i: (i, 0)),
        out_shape=jax.ShapeDtypeStruct((m, out_cols), out_dtype),
    )(table, w, b2d)


def _chunk_compute(pc_v, cid_v, len_v, rows_v, offs):
    def grp_body(tg, carry2):
        t0 = tg * VEC
        lens = len_v[pl.ds(t0, VEC)]
        for k in range(VEC):
            t = t0 + k
            nchars = lens[k]
            cvec = cid_v[t, :]
            accs = []
            for c in range(NGRP):
                accs.append(rows_v[t, pl.ds(32 * c, VEC)])
                accs.append(rows_v[t, pl.ds(32 * c + VEC, VEC)])
            accs = tuple(accs)

            def char_body(j, a):
                rv = cvec.at[jnp.full((VEC,), 0, jnp.int32) + j].get(
                    mode="promise_in_bounds")
                out = []
                for c in range(NGRP):
                    gi = plsc.load_gather(pc_v, [rv, offs[c]])
                    gb = plsc.bitcast(gi, jnp.bfloat16)
                    da, db = plsc.unpack(
                        gb, format=plsc.PackFormat.INTERLEAVED)
                    out.append(a[2 * c] + da)
                    out.append(a[2 * c + 1] + db)
                return tuple(out)

            accs = lax.fori_loop(0, nchars, char_body, accs)
            for c in range(NGRP):
                rows_v[t, pl.ds(32 * c, VEC)] = accs[2 * c]
                rows_v[t, pl.ds(32 * c + VEC, VEC)] = accs[2 * c + 1]
        return carry2

    lax.fori_loop(0, C // VEC, grp_body, 0)


@functools.partial(
    pl.kernel,
    out_type=jax.ShapeDtypeStruct((N, DOUT), jnp.float32),
    mesh=plsc.VectorSubcoreMesh(core_axis_name="c", subcore_axis_name="s"),
    scratch_types=[
        pltpu.VMEM((256, DOUT // 2), jnp.int32),
        pltpu.VMEM((C,), jnp.int32),
        pltpu.VMEM((C,), jnp.int32),
        pltpu.VMEM((C, MAXC), jnp.int32),
        pltpu.VMEM((C, MAXC), jnp.int32),
        pltpu.VMEM((C,), jnp.int32),
        pltpu.VMEM((C,), jnp.int32),
        pltpu.VMEM((C, DOUT), jnp.float32),
        pltpu.VMEM((C, DOUT), jnp.float32),
        pltpu.SemaphoreType.DMA,
        pltpu.SemaphoreType.DMA,
        pltpu.SemaphoreType.DMA,
        pltpu.SemaphoreType.DMA,
        pltpu.SemaphoreType.DMA,
        pltpu.SemaphoreType.DMA,
        pltpu.SemaphoreType.DMA,
        pltpu.SemaphoreType.DMA,
    ],
    compiler_params=pltpu.CompilerParams(needs_layout_passes=False),
)
def _sc_embed(pw_hbm, pc_hbm, widx_hbm, cid_hbm, len_hbm, out_hbm,
              pc_v, idx0, idx1, cid0, cid1, len0, len1,
              rows0, rows1, sem0, sem1, semc0, semc1,
              seml0, seml1, semw0, semw1):
    wid = lax.axis_index("s") * NC + lax.axis_index("c")
    base0 = wid * TPW
    pltpu.sync_copy(pc_hbm, pc_v)
    lane = lax.iota(jnp.int32, VEC)
    offs = [lane + VEC * c for c in range(NGRP)]

    def pair_body(i, carry):
        ba = base0 + 2 * i * C
        bb = ba + C
        pltpu.sync_copy(widx_hbm.at[pl.ds(ba, C)], idx0)
        h0 = pltpu.async_copy(pw_hbm.at[idx0], rows0, sem0)
        pltpu.sync_copy(widx_hbm.at[pl.ds(bb, C)], idx1)
        h1 = pltpu.async_copy(pw_hbm.at[idx1], rows1, sem1)
        hc0 = pltpu.async_copy(cid_hbm.at[pl.ds(ba, C)], cid0, semc0)
        hl0 = pltpu.async_copy(len_hbm.at[pl.ds(ba, C)], len0, seml0)
        hc1 = pltpu.async_copy(cid_hbm.at[pl.ds(bb, C)], cid1, semc1)
        hl1 = pltpu.async_copy(len_hbm.at[pl.ds(bb, C)], len1, seml1)
        h0.wait()
        hc0.wait()
        hl0.wait()
        _chunk_compute(pc_v, cid0, len0, rows0, offs)
        w0 = pltpu.async_copy(rows0, out_hbm.at[pl.ds(ba, C)], semw0)
        h1.wait()
        hc1.wait()
        hl1.wait()
        _chunk_compute(pc_v, cid1, len1, rows1, offs)
        w1 = pltpu.async_copy(rows1, out_hbm.at[pl.ds(bb, C)], semw1)
        w0.wait()
        w1.wait()
        return carry

    lax.fori_loop(0, NCHUNK // 2, pair_body, 0)


def kernel(word_inputs, char_ids, char_lengths, word_table, char_table, W, b):
    perm = jnp.array(_PERM, dtype=jnp.int32)
    wp = W[:, perm]
    pw = _project(word_table, W[:DW], b.reshape(1, DOUT), 2000,
                  _proj_f32_body, DOUT, jnp.float32)
    pc_bf = _project(char_table, wp[DW:], jnp.zeros((1, DOUT), jnp.float32),
                     256, _proj_bf16_body, DOUT, jnp.bfloat16)
    pc_i32 = lax.bitcast_convert_type(
        pc_bf.reshape(-1, DOUT // 2, 2), jnp.int32)
    widx = word_inputs.reshape(N).astype(jnp.int32)
    cid = char_ids.reshape(N, MAXC).astype(jnp.int32)
    clen = char_lengths.reshape(N).astype(jnp.int32)
    out = _sc_embed(pw, pc_i32, widx, cid, clen)
    return out.reshape(B, S, DOUT)


# speedup vs baseline: 2.2158x; 1.0090x over previous
import functools

import jax
import jax.numpy as jnp
from jax import lax
from jax.experimental import pallas as pl
from jax.experimental.pallas import tpu as pltpu
from jax.experimental.pallas import tpu_sc as plsc

B, S, MAXC = 1024, 200, 16
DW, DC, DOUT = 128, 64, 128
N = B * S
NC, NS = 2, 16
NW = NC * NS
TPW = N // NW
C = 128
NCHUNK = TPW // C
VEC = 16
NGRP = DOUT // 32

_PERM = tuple(
    32 * (p // 32) + (16 if p % 2 else 0) + (p % 32) // 2 for p in range(DOUT)
)


def _proj_f32_body(t_ref, w_ref, b_ref, out_ref):
    out_ref[...] = (
        jnp.dot(t_ref[...], w_ref[...], preferred_element_type=jnp.float32)
        + b_ref[...]
    )


def _proj_bf16_body(t_ref, w_ref, b_ref, out_ref):
    acc = (
        jnp.dot(t_ref[...], w_ref[...], preferred_element_type=jnp.float32)
        + b_ref[...]
    )
    out_ref[...] = acc.astype(jnp.bfloat16)


def _project(table, w, b2d, bm, body, out_cols, out_dtype):
    m, k = table.shape
    return pl.pallas_call(
        body,
        grid=(m // bm,),
        in_specs=[
            pl.BlockSpec((bm, k), lambda i: (i, 0)),
            pl.BlockSpec((k, DOUT), lambda i: (0, 0)),
            pl.BlockSpec((1, DOUT), lambda i: (0, 0)),
        ],
        out_specs=pl.BlockSpec((bm, out_cols), lambda i: (i, 0)),
        out_shape=jax.ShapeDtypeStruct((m, out_cols), out_dtype),
    )(table, w, b2d)


def _chunk_compute(pc_v, cid_v, len_v, rows_v, offs):
    def grp_body(tg, carry2):
        t0 = tg * VEC
        lens = len_v[pl.ds(t0, VEC)]
        for k in range(VEC):
            t = t0 + k
            nchars = lens[k]
            cvec = cid_v[t, :]
            accs = []
            for c in range(NGRP):
                accs.append(rows_v[t, pl.ds(32 * c, VEC)])
                accs.append(rows_v[t, pl.ds(32 * c + VEC, VEC)])
            accs = tuple(accs)

            def char_body(j, a):
                rv = cvec.at[jnp.full((VEC,), 0, jnp.int32) + j].get(
                    mode="promise_in_bounds")
                out = []
                for c in range(NGRP):
                    gi = plsc.load_gather(pc_v, [rv, offs[c]])
                    gb = plsc.bitcast(gi, jnp.bfloat16)
                    da, db = plsc.unpack(
                        gb, format=plsc.PackFormat.INTERLEAVED)
                    out.append(a[2 * c] + da)
                    out.append(a[2 * c + 1] + db)
                return tuple(out)

            accs = lax.fori_loop(0, nchars, char_body, accs)
            for c in range(NGRP):
                rows_v[t, pl.ds(32 * c, VEC)] = accs[2 * c]
                rows_v[t, pl.ds(32 * c + VEC, VEC)] = accs[2 * c + 1]
        return carry2

    lax.fori_loop(0, C // VEC, grp_body, 0)


@functools.partial(
    pl.kernel,
    out_type=jax.ShapeDtypeStruct((N, DOUT), jnp.float32),
    mesh=plsc.VectorSubcoreMesh(core_axis_name="c", subcore_axis_name="s"),
    scratch_types=[
        pltpu.VMEM((256, DOUT // 2), jnp.int32),
        pltpu.VMEM((TPW,), jnp.int32),
        pltpu.VMEM((TPW,), jnp.int32),
        pltpu.VMEM((C, MAXC), jnp.int32),
        pltpu.VMEM((C, MAXC), jnp.int32),
        pltpu.VMEM((C, DOUT), jnp.float32),
        pltpu.VMEM((C, DOUT), jnp.float32),
        pltpu.SemaphoreType.DMA,
        pltpu.SemaphoreType.DMA,
        pltpu.SemaphoreType.DMA,
        pltpu.SemaphoreType.DMA,
        pltpu.SemaphoreType.DMA,
        pltpu.SemaphoreType.DMA,
    ],
    compiler_params=pltpu.CompilerParams(needs_layout_passes=False),
)
def _sc_embed(pw_hbm, pc_hbm, widx_hbm, cid_hbm, len_hbm, out_hbm,
              pc_v, idx_all, len_all, cid0, cid1,
              rows0, rows1, sem0, sem1, semc0, semc1, semw0, semw1):
    wid = lax.axis_index("s") * NC + lax.axis_index("c")
    base0 = wid * TPW
    pltpu.sync_copy(pc_hbm, pc_v)
    pltpu.sync_copy(widx_hbm.at[pl.ds(base0, TPW)], idx_all)
    pltpu.sync_copy(len_hbm.at[pl.ds(base0, TPW)], len_all)
    lane = lax.iota(jnp.int32, VEC)
    offs = [lane + VEC * c for c in range(NGRP)]

    def pair_body(i, carry):
        oa = 2 * i * C
        ob = oa + C
        ba = base0 + oa
        bb = base0 + ob
        h0 = pltpu.async_copy(
            pw_hbm.at[idx_all.at[pl.ds(oa, C)]], rows0, sem0)
        h1 = pltpu.async_copy(
            pw_hbm.at[idx_all.at[pl.ds(ob, C)]], rows1, sem1)
        hc0 = pltpu.async_copy(cid_hbm.at[pl.ds(ba, C)], cid0, semc0)
        hc1 = pltpu.async_copy(cid_hbm.at[pl.ds(bb, C)], cid1, semc1)
        h0.wait()
        hc0.wait()
        _chunk_compute(pc_v, cid0, len_all.at[pl.ds(oa, C)], rows0, offs)
        w0 = pltpu.async_copy(rows0, out_hbm.at[pl.ds(ba, C)], semw0)
        h1.wait()
        hc1.wait()
        _chunk_compute(pc_v, cid1, len_all.at[pl.ds(ob, C)], rows1, offs)
        w1 = pltpu.async_copy(rows1, out_hbm.at[pl.ds(bb, C)], semw1)
        w0.wait()
        w1.wait()
        return carry

    lax.fori_loop(0, NCHUNK // 2, pair_body, 0)


def kernel(word_inputs, char_ids, char_lengths, word_table, char_table, W, b):
    perm = jnp.array(_PERM, dtype=jnp.int32)
    wp = W[:, perm]
    pw = _project(word_table, W[:DW], b.reshape(1, DOUT), 2000,
                  _proj_f32_body, DOUT, jnp.float32)
    pc_bf = _project(char_table, wp[DW:], jnp.zeros((1, DOUT), jnp.float32),
                     256, _proj_bf16_body, DOUT, jnp.bfloat16)
    pc_i32 = lax.bitcast_convert_type(
        pc_bf.reshape(-1, DOUT // 2, 2), jnp.int32)
    widx = word_inputs.reshape(N).astype(jnp.int32)
    cid = char_ids.reshape(N, MAXC).astype(jnp.int32)
    clen = char_lengths.reshape(N).astype(jnp.int32)
    out = _sc_embed(pw, pc_i32, widx, cid, clen)
    return out.reshape(B, S, DOUT)


# speedup vs baseline: 2.2849x; 1.0312x over previous
import functools

import jax
import jax.numpy as jnp
from jax import lax
from jax.experimental import pallas as pl
from jax.experimental.pallas import tpu as pltpu
from jax.experimental.pallas import tpu_sc as plsc

B, S, MAXC = 1024, 200, 16
DW, DC, DOUT = 128, 64, 128
N = B * S
NC, NS = 2, 16
NW = NC * NS
TPW = N // NW
C = 128
NCHUNK = TPW // C
VEC = 16
NGRP = DOUT // 32

_PERM = tuple(
    32 * (p // 32) + (16 if p % 2 else 0) + (p % 32) // 2 for p in range(DOUT)
)


def _proj_f32_body(t_ref, w_ref, b_ref, out_ref):
    out_ref[...] = (
        jnp.dot(t_ref[...], w_ref[...], preferred_element_type=jnp.float32)
        + b_ref[...]
    )


def _proj_bf16_body(t_ref, w_ref, b_ref, out_ref):
    acc = (
        jnp.dot(t_ref[...], w_ref[...], preferred_element_type=jnp.float32)
        + b_ref[...]
    )
    out_ref[...] = acc.astype(jnp.bfloat16)


def _project(table, w, b2d, bm, body, out_cols, out_dtype):
    m, k = table.shape
    return pl.pallas_call(
        body,
        grid=(m // bm,),
        in_specs=[
            pl.BlockSpec((bm, k), lambda i: (i, 0)),
            pl.BlockSpec((k, DOUT), lambda i: (0, 0)),
            pl.BlockSpec((1, DOUT), lambda i: (0, 0)),
        ],
        out_specs=pl.BlockSpec((bm, out_cols), lambda i: (i, 0)),
        out_shape=jax.ShapeDtypeStruct((m, out_cols), out_dtype),
    )(table, w, b2d)


def _chunk_compute(pc_v, cid_v, len_v, rows_v, offs):
    def grp_body(tg, carry2):
        t0 = tg * VEC
        lens = len_v[pl.ds(t0, VEC)]
        for k in range(VEC):
            t = t0 + k
            nchars = lens[k]
            cvec = cid_v[t, :]
            accs = []
            for c in range(NGRP):
                accs.append(rows_v[t, pl.ds(32 * c, VEC)])
                accs.append(rows_v[t, pl.ds(32 * c + VEC, VEC)])
            accs = tuple(accs)

            def char_body(j, a):
                rv = cvec.at[jnp.full((VEC,), 0, jnp.int32) + j].get(
                    mode="promise_in_bounds")
                out = []
                for c in range(NGRP):
                    gi = plsc.load_gather(pc_v, [rv, offs[c]])
                    gb = plsc.bitcast(gi, jnp.bfloat16)
                    da, db = plsc.unpack(
                        gb, format=plsc.PackFormat.INTERLEAVED)
                    out.append(a[2 * c] + da)
                    out.append(a[2 * c + 1] + db)
                return tuple(out)

            accs = lax.fori_loop(0, nchars, char_body, accs)
            for c in range(NGRP):
                rows_v[t, pl.ds(32 * c, VEC)] = accs[2 * c]
                rows_v[t, pl.ds(32 * c + VEC, VEC)] = accs[2 * c + 1]
        return carry2

    lax.fori_loop(0, C // VEC, grp_body, 0)


@functools.partial(
    pl.kernel,
    out_type=jax.ShapeDtypeStruct((N, DOUT), jnp.float32),
    mesh=plsc.VectorSubcoreMesh(core_axis_name="c", subcore_axis_name="s"),
    scratch_types=[
        pltpu.VMEM((256, DOUT // 2), jnp.int32),
        pltpu.VMEM((TPW,), jnp.int32),
        pltpu.VMEM((TPW,), jnp.int32),
        pltpu.VMEM((C, MAXC), jnp.int32),
        pltpu.VMEM((C, MAXC), jnp.int32),
        pltpu.VMEM((C, DOUT), jnp.float32),
        pltpu.VMEM((C, DOUT), jnp.float32),
        pltpu.SemaphoreType.DMA,
        pltpu.SemaphoreType.DMA,
        pltpu.SemaphoreType.DMA,
        pltpu.SemaphoreType.DMA,
        pltpu.SemaphoreType.DMA,
        pltpu.SemaphoreType.DMA,
    ],
    compiler_params=pltpu.CompilerParams(needs_layout_passes=False),
)
def _sc_embed(pw_hbm, pc_hbm, widx_hbm, cid_hbm, len_hbm, out_hbm,
              pc_v, idx_all, len_all, cid0, cid1,
              rows0, rows1, sem0, sem1, semc0, semc1, semw0, semw1):
    wid = lax.axis_index("s") * NC + lax.axis_index("c")
    base0 = wid * TPW
    pltpu.sync_copy(pc_hbm, pc_v)
    pltpu.sync_copy(widx_hbm.at[pl.ds(base0, TPW)], idx_all)
    pltpu.sync_copy(len_hbm.at[pl.ds(base0, TPW)], len_all)
    lane = lax.iota(jnp.int32, VEC)
    offs = [lane + VEC * c for c in range(NGRP)]

    pltpu.async_copy(pw_hbm.at[idx_all.at[pl.ds(0, C)]], rows0, sem0)

    def pair_body(i, carry):
        oa = 2 * i * C
        ob = oa + C
        ba = base0 + oa
        bb = base0 + ob
        h1 = pltpu.async_copy(
            pw_hbm.at[idx_all.at[pl.ds(ob, C)]], rows1, sem1)
        hc0 = pltpu.async_copy(cid_hbm.at[pl.ds(ba, C)], cid0, semc0)
        hc1 = pltpu.async_copy(cid_hbm.at[pl.ds(bb, C)], cid1, semc1)
        pltpu.make_async_copy(
            pw_hbm.at[idx_all.at[pl.ds(oa, C)]], rows0, sem0).wait()
        hc0.wait()
        _chunk_compute(pc_v, cid0, len_all.at[pl.ds(oa, C)], rows0, offs)
        w0 = pltpu.async_copy(rows0, out_hbm.at[pl.ds(ba, C)], semw0)
        h1.wait()
        hc1.wait()
        _chunk_compute(pc_v, cid1, len_all.at[pl.ds(ob, C)], rows1, offs)
        w1 = pltpu.async_copy(rows1, out_hbm.at[pl.ds(bb, C)], semw1)
        w0.wait()
        onext = jnp.minimum(oa + 2 * C, (NCHUNK - 2) * C)
        pltpu.async_copy(pw_hbm.at[idx_all.at[pl.ds(onext, C)]], rows0, sem0)
        w1.wait()
        return carry

    lax.fori_loop(0, NCHUNK // 2, pair_body, 0)
    pltpu.make_async_copy(
        pw_hbm.at[idx_all.at[pl.ds((NCHUNK - 2) * C, C)]], rows0, sem0).wait()


def kernel(word_inputs, char_ids, char_lengths, word_table, char_table, W, b):
    perm = jnp.array(_PERM, dtype=jnp.int32)
    wp = W[:, perm]
    pw = _project(word_table, W[:DW], b.reshape(1, DOUT), 2000,
                  _proj_f32_body, DOUT, jnp.float32)
    pc_bf = _project(char_table, wp[DW:], jnp.zeros((1, DOUT), jnp.float32),
                     256, _proj_bf16_body, DOUT, jnp.bfloat16)
    pc_i32 = lax.bitcast_convert_type(
        pc_bf.reshape(-1, DOUT // 2, 2), jnp.int32)
    widx = word_inputs.reshape(N).astype(jnp.int32)
    cid = char_ids.reshape(N, MAXC).astype(jnp.int32)
    clen = char_lengths.reshape(N).astype(jnp.int32)
    out = _sc_embed(pw, pc_i32, widx, cid, clen)
    return out.reshape(B, S, DOUT)
